# baseline (speedup 1.0000x reference)
import functools

import jax
import jax.numpy as jnp
from jax.experimental import pallas as pl
from jax.experimental.pallas import tpu as pltpu

NUM_Q = 4
N_E = 1024
E_DIM = 128
N_TOK = 16384
BETA = 0.55
EPS = 1e-12

TILE = 2048
LOSS_SCALE = (1.0 + BETA) / (NUM_Q * N_TOK * E_DIM)


def _rvq_kernel(x_ref, cb_ref, xq_ref, loss_ref,
                i0_ref, i1_ref, i2_ref, i3_ref,
                r0_ref, r1_ref, r2_ref, r3_ref):
    step = pl.program_id(0)
    nsteps = pl.num_programs(0)

    @pl.when(step == 0)
    def _init():
        loss_ref[...] = jnp.zeros_like(loss_ref)

    resid = x_ref[...]
    xq = jnp.zeros_like(resid)
    loss_part = jnp.zeros((1, E_DIM), dtype=jnp.float32)
    idx_refs = (i0_ref, i1_ref, i2_ref, i3_ref)
    res_refs = (r0_ref, r1_ref, r2_ref, r3_ref)
    lane = jax.lax.broadcasted_iota(jnp.int32, (TILE, N_E), 1)

    for q_i in range(NUM_Q):
        n = jnp.sqrt(jnp.sum(resid * resid, axis=-1, keepdims=True))
        z = resid / jnp.maximum(n, EPS)
        zz = jnp.sum(z * z, axis=-1, keepdims=True)
        ee = jnp.sum(cb_ref[q_i] * cb_ref[q_i], axis=-1)[None, :]
        s = jax.lax.dot_general(z.astype(jnp.bfloat16),
                                cb_ref[q_i].astype(jnp.bfloat16),
                                (((1,), (1,)), ((), ())),
                                preferred_element_type=jnp.float32)
        d = zz - 2.0 * s + ee
        dmin = jnp.min(d, axis=-1, keepdims=True)
        idx = jnp.min(jnp.where(d == dmin, lane, N_E), axis=-1, keepdims=True)
        idx_refs[q_i][...] = idx
        oh = (lane == idx).astype(jnp.float32)
        q = jax.lax.dot_general(oh, cb_ref[q_i], (((1,), (0,)), ((), ())),
                                precision=jax.lax.Precision.HIGHEST,
                                preferred_element_type=jnp.float32)
        dq = q - z
        loss_part = loss_part + jnp.sum(dq * dq, axis=0, keepdims=True)
        x_res = z + dq
        resid = resid - x_res
        xq = xq + x_res
        res_refs[q_i][...] = resid

    xq_ref[...] = xq
    loss_ref[...] += loss_part

    @pl.when(step == nsteps - 1)
    def _final():
        loss_ref[...] = jnp.full((1, E_DIM), jnp.sum(loss_ref[...]) * LOSS_SCALE,
                                 dtype=jnp.float32)


@jax.jit
def kernel(x, codebooks):
    grid = (N_TOK // TILE,)
    tok_spec = pl.BlockSpec((TILE, E_DIM), lambda i: (i, 0))
    idx_spec = pl.BlockSpec((TILE, 1), lambda i: (i, 0))
    out_shapes = (
        jax.ShapeDtypeStruct((N_TOK, E_DIM), jnp.float32),
        jax.ShapeDtypeStruct((1, E_DIM), jnp.float32),
        *[jax.ShapeDtypeStruct((N_TOK, 1), jnp.int32) for _ in range(NUM_Q)],
        *[jax.ShapeDtypeStruct((N_TOK, E_DIM), jnp.float32) for _ in range(NUM_Q)],
    )
    out_specs = (
        tok_spec,
        pl.BlockSpec((1, E_DIM), lambda i: (0, 0)),
        *[idx_spec] * NUM_Q,
        *[tok_spec] * NUM_Q,
    )
    outs = pl.pallas_call(
        _rvq_kernel,
        grid=grid,
        in_specs=[
            tok_spec,
            pl.BlockSpec((NUM_Q, N_E, E_DIM), lambda i: (0, 0, 0)),
        ],
        out_specs=out_specs,
        out_shape=out_shapes,
    )(x, codebooks)
    x_q, loss_acc = outs[0], outs[1]
    idxs = outs[2:2 + NUM_Q]
    resids = outs[2 + NUM_Q:]
    mean_losses = loss_acc[0, 0]
    all_indices = jnp.concatenate(idxs, axis=1)
    return (x_q, mean_losses, all_indices, tuple(resids))


# speedup vs baseline: 1.8960x; 1.8960x over previous
import functools

import jax
import jax.numpy as jnp
from jax.experimental import pallas as pl
from jax.experimental.pallas import tpu as pltpu

NUM_Q = 4
N_E = 1024
E_DIM = 128
N_TOK = 16384
BETA = 0.55
EPS = 1e-12

TILE = 2048
LOSS_SCALE = (1.0 + BETA) / (NUM_Q * N_TOK * E_DIM)


def _rvq_kernel(x_ref, cb_ref, xq_ref, loss_ref,
                i0_ref, i1_ref, i2_ref, i3_ref,
                r0_ref, r1_ref, r2_ref, r3_ref):
    step = pl.program_id(0)
    nsteps = pl.num_programs(0)

    @pl.when(step == 0)
    def _init():
        loss_ref[...] = jnp.zeros_like(loss_ref)

    resid = x_ref[...]
    xq = jnp.zeros_like(resid)
    loss_part = jnp.zeros((1, E_DIM), dtype=jnp.float32)
    idx_refs = (i0_ref, i1_ref, i2_ref, i3_ref)
    res_refs = (r0_ref, r1_ref, r2_ref, r3_ref)
    lane = jax.lax.broadcasted_iota(jnp.int32, (TILE, N_E), 1)

    for q_i in range(NUM_Q):
        n = jnp.sqrt(jnp.sum(resid * resid, axis=-1, keepdims=True))
        z = resid / jnp.maximum(n, EPS)
        zz = jnp.sum(z * z, axis=-1, keepdims=True)
        ee = jnp.sum(cb_ref[q_i] * cb_ref[q_i], axis=-1)[None, :]
        s = jax.lax.dot_general(z.astype(jnp.bfloat16),
                                cb_ref[q_i].astype(jnp.bfloat16),
                                (((1,), (1,)), ((), ())),
                                preferred_element_type=jnp.float32)
        d = zz - 2.0 * s + ee
        dmin = jnp.min(d, axis=-1, keepdims=True)
        idx = jnp.min(jnp.where(d == dmin, lane, N_E), axis=-1, keepdims=True)
        idx_refs[q_i][...] = idx
        E = cb_ref[q_i]
        e_hi = E.astype(jnp.bfloat16)
        rem1 = E - e_hi.astype(jnp.float32)
        e_mid = rem1.astype(jnp.bfloat16)
        e_lo = (rem1 - e_mid.astype(jnp.float32)).astype(jnp.bfloat16)
        oh = (lane == idx).astype(jnp.bfloat16)
        dn = (((1,), (0,)), ((), ()))
        qh = jax.lax.dot_general(oh, e_hi, dn, preferred_element_type=jnp.float32)
        qm = jax.lax.dot_general(oh, e_mid, dn, preferred_element_type=jnp.float32)
        ql = jax.lax.dot_general(oh, e_lo, dn, preferred_element_type=jnp.float32)
        q = (qh + qm) + ql
        dq = q - z
        loss_part = loss_part + jnp.sum(dq * dq, axis=0, keepdims=True)
        x_res = z + dq
        resid = resid - x_res
        xq = xq + x_res
        res_refs[q_i][...] = resid

    xq_ref[...] = xq
    loss_ref[...] += loss_part

    @pl.when(step == nsteps - 1)
    def _final():
        loss_ref[...] = jnp.full((1, E_DIM), jnp.sum(loss_ref[...]) * LOSS_SCALE,
                                 dtype=jnp.float32)


@jax.jit
def kernel(x, codebooks):
    grid = (N_TOK // TILE,)
    tok_spec = pl.BlockSpec((TILE, E_DIM), lambda i: (i, 0))
    idx_spec = pl.BlockSpec((TILE, 1), lambda i: (i, 0))
    out_shapes = (
        jax.ShapeDtypeStruct((N_TOK, E_DIM), jnp.float32),
        jax.ShapeDtypeStruct((1, E_DIM), jnp.float32),
        *[jax.ShapeDtypeStruct((N_TOK, 1), jnp.int32) for _ in range(NUM_Q)],
        *[jax.ShapeDtypeStruct((N_TOK, E_DIM), jnp.float32) for _ in range(NUM_Q)],
    )
    out_specs = (
        tok_spec,
        pl.BlockSpec((1, E_DIM), lambda i: (0, 0)),
        *[idx_spec] * NUM_Q,
        *[tok_spec] * NUM_Q,
    )
    outs = pl.pallas_call(
        _rvq_kernel,
        grid=grid,
        in_specs=[
            tok_spec,
            pl.BlockSpec((NUM_Q, N_E, E_DIM), lambda i: (0, 0, 0)),
        ],
        out_specs=out_specs,
        out_shape=out_shapes,
    )(x, codebooks)
    x_q, loss_acc = outs[0], outs[1]
    idxs = outs[2:2 + NUM_Q]
    resids = outs[2 + NUM_Q:]
    mean_losses = loss_acc[0, 0]
    all_indices = jnp.concatenate(idxs, axis=1)
    return (x_q, mean_losses, all_indices, tuple(resids))


# speedup vs baseline: 3.3570x; 1.7705x over previous
import functools

import jax
import jax.numpy as jnp
from jax.experimental import pallas as pl
from jax.experimental.pallas import tpu as pltpu

NUM_Q = 4
N_E = 1024
E_DIM = 128
N_TOK = 16384
BETA = 0.55
EPS = 1e-12

TILE = 2048
LOSS_SCALE = (1.0 + BETA) / (NUM_Q * N_TOK * E_DIM)


def _rvq_kernel(x_ref, cb_ref, xq_ref, loss_ref,
                i0_ref, i1_ref, i2_ref, i3_ref,
                r0_ref, r1_ref, r2_ref, r3_ref):
    step = pl.program_id(0)
    nsteps = pl.num_programs(0)

    @pl.when(step == 0)
    def _init():
        loss_ref[...] = jnp.zeros_like(loss_ref)

    resid = x_ref[...]
    xq = jnp.zeros_like(resid)
    loss_part = jnp.zeros((1, E_DIM), dtype=jnp.float32)
    idx_refs = (i0_ref, i1_ref, i2_ref, i3_ref)
    res_refs = (r0_ref, r1_ref, r2_ref, r3_ref)
    lane = jax.lax.broadcasted_iota(jnp.int32, (TILE, N_E), 1)

    for q_i in range(NUM_Q):
        n = jnp.sqrt(jnp.sum(resid * resid, axis=-1, keepdims=True))
        z = resid / jnp.maximum(n, EPS)
        zz = jnp.sum(z * z, axis=-1, keepdims=True)
        ee = jnp.sum(cb_ref[q_i] * cb_ref[q_i], axis=-1)[None, :]
        s = jax.lax.dot_general(z.astype(jnp.bfloat16),
                                cb_ref[q_i].astype(jnp.bfloat16),
                                (((1,), (1,)), ((), ())),
                                preferred_element_type=jnp.float32)
        d = zz - 2.0 * s + ee
        dmin = jnp.min(d, axis=-1, keepdims=True)
        idx = jnp.min(jnp.where(d == dmin, lane, N_E), axis=-1, keepdims=True)
        idx_refs[q_i][...] = idx
        E = cb_ref[q_i]
        e_hi = E.astype(jnp.bfloat16)
        rem1 = E - e_hi.astype(jnp.float32)
        e_mid = rem1.astype(jnp.bfloat16)
        e_lo = (rem1 - e_mid.astype(jnp.float32)).astype(jnp.bfloat16)
        e_cat = jnp.concatenate([e_hi, e_mid, e_lo], axis=1)
        oh = (lane == idx).astype(jnp.bfloat16)
        dn = (((1,), (0,)), ((), ()))
        q3 = jax.lax.dot_general(oh, e_cat, dn, preferred_element_type=jnp.float32)
        q = (q3[:, :E_DIM] + q3[:, E_DIM:2 * E_DIM]) + q3[:, 2 * E_DIM:]
        dq = q - z
        loss_part = loss_part + jnp.sum(dq * dq, axis=0, keepdims=True)
        x_res = z + dq
        resid = resid - x_res
        xq = xq + x_res
        res_refs[q_i][...] = resid

    xq_ref[...] = xq
    loss_ref[...] += loss_part

    @pl.when(step == nsteps - 1)
    def _final():
        loss_ref[...] = jnp.full((1, E_DIM), jnp.sum(loss_ref[...]) * LOSS_SCALE,
                                 dtype=jnp.float32)


@jax.jit
def kernel(x, codebooks):
    grid = (N_TOK // TILE,)
    tok_spec = pl.BlockSpec((TILE, E_DIM), lambda i: (i, 0))
    idx_spec = pl.BlockSpec((TILE, 1), lambda i: (i, 0))
    out_shapes = (
        jax.ShapeDtypeStruct((N_TOK, E_DIM), jnp.float32),
        jax.ShapeDtypeStruct((1, E_DIM), jnp.float32),
        *[jax.ShapeDtypeStruct((N_TOK, 1), jnp.int32) for _ in range(NUM_Q)],
        *[jax.ShapeDtypeStruct((N_TOK, E_DIM), jnp.float32) for _ in range(NUM_Q)],
    )
    out_specs = (
        tok_spec,
        pl.BlockSpec((1, E_DIM), lambda i: (0, 0)),
        *[idx_spec] * NUM_Q,
        *[tok_spec] * NUM_Q,
    )
    outs = pl.pallas_call(
        _rvq_kernel,
        grid=grid,
        in_specs=[
            tok_spec,
            pl.BlockSpec((NUM_Q, N_E, E_DIM), lambda i: (0, 0, 0)),
        ],
        out_specs=out_specs,
        out_shape=out_shapes,
    )(x, codebooks)
    x_q, loss_acc = outs[0], outs[1]
    idxs = outs[2:2 + NUM_Q]
    resids = outs[2 + NUM_Q:]
    mean_losses = loss_acc[0, 0]
    all_indices = jnp.concatenate(idxs, axis=1)
    return (x_q, mean_losses, all_indices, tuple(resids))
